# Initial kernel scaffold; baseline (speedup 1.0000x reference)
import functools
import jax
import jax.numpy as jnp
from jax import lax
from jax.experimental import pallas as pl
from jax.experimental.pallas import tpu as pltpu
from jax.experimental.pallas import tpu_sc as plsc

N = 10000
CH = 12
CP = 16
NP = N + 16
RPS = NP // 16
EPS = 1e-5
NC, NS = 2, 16
CHUNK = 128
SLAB = 8
BQ = 400

_sc_mesh = functools.partial(
    plsc.VectorSubcoreMesh, core_axis_name="c", subcore_axis_name="s")


def _sc_degree(ei_chunks, ones_rows, zeros_acc):
  nchunks = ei_chunks.shape[0]
  per_tile = nchunks // (NC * NS)
  outer = per_tile // SLAB

  @functools.partial(
      pl.kernel,
      mesh=_sc_mesh(),
      out_type=jax.ShapeDtypeStruct((NC, NP, CP), jnp.float32),
      scratch_types=[
          pltpu.VMEM((SLAB, 2, CHUNK), jnp.int32),
          pltpu.VMEM((CHUNK, CP), jnp.float32),
          pltpu.VMEM_SHARED((NP, CP), jnp.float32),
      ],
  )
  def deg_kernel(ei_hbm, ones_hbm, zeros_hbm, out_hbm, idx_v, ones_v, acc):
    c = lax.axis_index("c")
    s = lax.axis_index("s")
    wid = c * NS + s
    pltpu.sync_copy(zeros_hbm.at[pl.ds(s * RPS, RPS)],
                    acc.at[pl.ds(s * RPS, RPS)])
    pltpu.sync_copy(ones_hbm, ones_v)
    plsc.subcore_barrier()

    def body(i, carry):
      base = wid * per_tile + i * SLAB
      pltpu.sync_copy(ei_hbm.at[pl.ds(base, SLAB)], idx_v)
      for j in range(SLAB):
        pltpu.sync_copy(ones_v, acc.at[idx_v.at[j, 1]], add=True)
      return carry

    lax.fori_loop(0, outer, body, 0)
    plsc.subcore_barrier()
    pltpu.sync_copy(acc.at[pl.ds(s * RPS, RPS)],
                    out_hbm.at[c, pl.ds(s * RPS, RPS)])

  return deg_kernel(ei_chunks, ones_rows, zeros_acc)


def _sc_edge_agg(ei_chunks, g, zeros_acc):
  nchunks = ei_chunks.shape[0]
  per_tile = nchunks // (NC * NS)
  outer = per_tile // SLAB

  @functools.partial(
      pl.kernel,
      mesh=_sc_mesh(),
      out_type=jax.ShapeDtypeStruct((NC, NP, CP), jnp.float32),
      scratch_types=[
          pltpu.VMEM((SLAB, 2, CHUNK), jnp.int32),
          pltpu.VMEM((SLAB, CHUNK, CP), jnp.float32),
          pltpu.VMEM_SHARED((NP, CP), jnp.float32),
          pltpu.SemaphoreType.DMA,
      ],
  )
  def agg_kernel(ei_hbm, g_hbm, zeros_hbm, out_hbm, idx_v, rows_v, acc, sem):
    c = lax.axis_index("c")
    s = lax.axis_index("s")
    wid = c * NS + s
    pltpu.sync_copy(zeros_hbm.at[pl.ds(s * RPS, RPS)],
                    acc.at[pl.ds(s * RPS, RPS)])
    plsc.subcore_barrier()

    def body(i, carry):
      base = wid * per_tile + i * SLAB
      pltpu.sync_copy(ei_hbm.at[pl.ds(base, SLAB)], idx_v)
      copies = [
          pltpu.async_copy(g_hbm.at[idx_v.at[j, 0]], rows_v.at[j], sem)
          for j in range(SLAB)
      ]
      for j in range(SLAB):
        copies[j].wait()
      for j in range(SLAB):
        pltpu.sync_copy(rows_v.at[j], acc.at[idx_v.at[j, 1]], add=True)
      return carry

    lax.fori_loop(0, outer, body, 0)
    plsc.subcore_barrier()
    pltpu.sync_copy(acc.at[pl.ds(s * RPS, RPS)],
                    out_hbm.at[c, pl.ds(s * RPS, RPS)])

  return agg_kernel(ei_chunks, g, zeros_acc)


def _fs(shape):
  return pl.BlockSpec(shape, lambda: (0,) * len(shape))


def _bn(h, g, b, m, v):
  return (h - m) * lax.rsqrt(v + EPS) * g + b


def _tc_input(x, Win, b_in, cnt, W0):
  def body(x_ref, win_ref, bin_ref, cnt_ref, w0_ref, h_ref, g_ref, dinv_ref):
    h = jnp.dot(x_ref[...], win_ref[...],
                preferred_element_type=jnp.float32) + bin_ref[...]
    deg = 1.0 + cnt_ref[0, :N, :] + cnt_ref[1, :N, :]
    dinv = lax.rsqrt(deg)
    dinv_ref[...] = dinv
    hW = jnp.dot(h, w0_ref[...], preferred_element_type=jnp.float32)
    gg = hW * dinv[:, :CH]
    g_ref[...] = jnp.concatenate(
        [gg, jnp.zeros((N, CP - CH), jnp.float32)], axis=1)
    h_ref[...] = h

  return pl.pallas_call(
      body,
      out_shape=[
          jax.ShapeDtypeStruct((N, CH), jnp.float32),
          jax.ShapeDtypeStruct((N, CP), jnp.float32),
          jax.ShapeDtypeStruct((N, CP), jnp.float32),
      ],
      in_specs=[_fs(x.shape), _fs(Win.shape), _fs(b_in.shape),
                _fs(cnt.shape), _fs(W0.shape)],
      out_specs=[_fs((N, CH)), _fs((N, CP)), _fs((N, CP))],
  )(x, Win, b_in, cnt, W0)


def _tc_attention(h, p):
  nblk = N // BQ

  def body(hq_ref, h_ref, wq, bq, wk, bk, wv, bv, wo, bo,
           g2, b2, m2, v2, out_ref, k_s, v_s):
    i = pl.program_id(0)

    @pl.when(i == 0)
    def _():
      hf = h_ref[...]
      k_s[...] = jnp.dot(hf, wk[...],
                         preferred_element_type=jnp.float32) + bk[...]
      v_s[...] = jnp.dot(hf, wv[...],
                         preferred_element_type=jnp.float32) + bv[...]

    hq = hq_ref[...]
    q = jnp.dot(hq, wq[...], preferred_element_type=jnp.float32) + bq[...]
    s = lax.dot_general(q, k_s[...], (((1,), (1,)), ((), ())),
                        preferred_element_type=jnp.float32)
    s = s * (1.0 / jnp.sqrt(float(CH)))
    mx = jnp.max(s, axis=1, keepdims=True)
    ex = jnp.exp(s - mx)
    den = jnp.sum(ex, axis=1, keepdims=True)
    o = jnp.dot(ex, v_s[...], preferred_element_type=jnp.float32) / den
    hg = jnp.dot(o, wo[...], preferred_element_type=jnp.float32) \
        + bo[...] + hq
    out_ref[...] = _bn(hg, g2[...], b2[...], m2[...], v2[...])

  params = [p['Wq'], p['bq'], p['Wk'], p['bk'], p['Wv'], p['bv'],
            p['Wo'], p['bo'], p['bn2_g'], p['bn2_b'], p['bn2_m'], p['bn2_v']]

  def cspec(a):
    sh = a.shape
    return pl.BlockSpec(sh, lambda i: (0,) * len(sh))

  return pl.pallas_call(
      body,
      grid=(nblk,),
      out_shape=jax.ShapeDtypeStruct((N, CH), jnp.float32),
      in_specs=[pl.BlockSpec((BQ, CH), lambda i: (i, 0)),
                pl.BlockSpec((N, CH), lambda i: (0, 0))] +
               [cspec(a) for a in params],
      out_specs=pl.BlockSpec((BQ, CH), lambda i: (i, 0)),
      scratch_shapes=[pltpu.VMEM((N, CH), jnp.float32),
                      pltpu.VMEM((N, CH), jnp.float32)],
  )(h, h, *params)


def _tc_combine(h, part, hg, dinv, p, lng, lnb, W_next, b_next, last):
  def body(h_ref, part_ref, hg_ref, dinv_ref, gw, gb,
           g1, b1_, m1, v1, w1, bb1, w2, bb2,
           g3, b3_, m3, v3, lng_ref, lnb_ref, wn, bn_, *outs):
    h0 = h_ref[...]
    dinv = dinv_ref[...][:, :CH]
    hW = jnp.dot(h0, gw[...], preferred_element_type=jnp.float32)
    e = part_ref[0, :N, :CH] + part_ref[1, :N, :CH]
    hl = e * dinv + hW * (dinv * dinv) + gb[...] + h0
    hl = _bn(hl, g1[...], b1_[...], m1[...], v1[...])
    out = hl + hg_ref[...]
    mlp = jnp.maximum(
        jnp.dot(out, w1[...], preferred_element_type=jnp.float32) + bb1[...],
        0.0)
    mlp = jnp.dot(mlp, w2[...], preferred_element_type=jnp.float32) + bb2[...]
    out = out + mlp
    out = _bn(out, g3[...], b3_[...], m3[...], v3[...])
    mu = jnp.sum(out, axis=1, keepdims=True) * (1.0 / CH)
    d = out - mu
    var = jnp.sum(d * d, axis=1, keepdims=True) * (1.0 / CH)
    hn = jnp.maximum(d * lax.rsqrt(var + EPS) * lng_ref[...] + lnb_ref[...],
                     0.0)
    if last:
      logits = jnp.dot(hn, wn[...], preferred_element_type=jnp.float32) \
          + bn_[...]
      mx = jnp.max(logits, axis=1, keepdims=True)
      z = logits - mx
      lse = jnp.log(jnp.sum(jnp.exp(z), axis=1, keepdims=True))
      outs[0][...] = z - lse
    else:
      outs[0][...] = hn
      gg = jnp.dot(hn, wn[...], preferred_element_type=jnp.float32) * dinv
      outs[1][...] = jnp.concatenate(
          [gg, jnp.zeros((N, CP - CH), jnp.float32)], axis=1)

  ins = [h, part, hg, dinv, p['gcn_W'], p['gcn_b'],
         p['bn1_g'], p['bn1_b'], p['bn1_m'], p['bn1_v'],
         p['W1'], p['b1'], p['W2'], p['b2'],
         p['bn3_g'], p['bn3_b'], p['bn3_m'], p['bn3_v'],
         lng, lnb, W_next, b_next]
  if last:
    out_shape = [jax.ShapeDtypeStruct((N, W_next.shape[1]), jnp.float32)]
  else:
    out_shape = [jax.ShapeDtypeStruct((N, CH), jnp.float32),
                 jax.ShapeDtypeStruct((N, CP), jnp.float32)]
  return pl.pallas_call(
      body,
      out_shape=out_shape,
      in_specs=[_fs(a.shape) for a in ins],
      out_specs=[_fs(o.shape) for o in out_shape],
  )(*ins)


def kernel(x, edge_index, Win, b_in, params, Wout, b_out, ln_g, ln_b):
  ei = edge_index.astype(jnp.int32)
  E = ei.shape[1]
  group = NC * NS * CHUNK * SLAB
  EP = ((E + group - 1) // group) * group
  pad = EP - E
  src = jnp.concatenate([ei[0], jnp.zeros((pad,), jnp.int32)])
  dst = jnp.concatenate([ei[1], jnp.full((pad,), N, jnp.int32)])
  ei_chunks = jnp.stack(
      [src.reshape(-1, CHUNK), dst.reshape(-1, CHUNK)], axis=1)

  ones_rows = jnp.ones((CHUNK, CP), jnp.float32)
  zeros_acc = jnp.zeros((NP, CP), jnp.float32)

  def r2(v):
    return v.reshape(1, -1)

  cnt = _sc_degree(ei_chunks, ones_rows, zeros_acc)
  h0, g0, dinv = _tc_input(x, Win, r2(b_in), cnt, params['gcn_W'][0])

  h = h0
  g = g0
  for l in range(2):
    p = dict((k, v[l]) for k, v in params.items())
    for k in list(p):
      if p[k].ndim == 1:
        p[k] = r2(p[k])
    part = _sc_edge_agg(ei_chunks, g, zeros_acc)
    hg = _tc_attention(h, p)
    last = (l == 1)
    if last:
      res = _tc_combine(h, part, hg, dinv, p, r2(ln_g[l]), r2(ln_b[l]),
                        Wout, r2(b_out), True)
      return res[0]
    nxt = _tc_combine(h, part, hg, dinv, p, r2(ln_g[l]), r2(ln_b[l]),
                      params['gcn_W'][l + 1], r2(params['gcn_b'][l + 1]),
                      False)
    h, g = nxt[0], nxt[1]


# speedup vs baseline: 13.1123x; 13.1123x over previous
import functools
import jax
import jax.numpy as jnp
from jax import lax
from jax.experimental import pallas as pl
from jax.experimental.pallas import tpu as pltpu
from jax.experimental.pallas import tpu_sc as plsc

N = 10000
CH = 12
CP = 16
NP = N + 112
RPS = NP // 16
EPS = 1e-5
NC, NS = 2, 16
CHUNK = 128
SLAB = 8
BQ = 400

_sc_mesh = functools.partial(
    plsc.VectorSubcoreMesh, core_axis_name="c", subcore_axis_name="s")


def _sc_degree(ei_chunks, ones_rows, zeros_acc):
  nchunks = ei_chunks.shape[0]
  per_tile = nchunks // (NC * NS)
  outer = per_tile // SLAB

  @functools.partial(
      pl.kernel,
      mesh=_sc_mesh(),
      out_type=jax.ShapeDtypeStruct((NC, NP, CP), jnp.float32),
      compiler_params=pltpu.CompilerParams(use_tc_tiling_on_sc=False),
      scratch_types=[
          pltpu.VMEM((SLAB, 2, CHUNK), jnp.int32),
          pltpu.VMEM((CHUNK, CP), jnp.float32),
          pltpu.VMEM_SHARED((NP, CP), jnp.float32),
      ],
  )
  def deg_kernel(ei_hbm, ones_hbm, zeros_hbm, out_hbm, idx_v, ones_v, acc):
    c = lax.axis_index("c")
    s = lax.axis_index("s")
    wid = c * NS + s
    pltpu.sync_copy(zeros_hbm.at[pl.ds(s * RPS, RPS)],
                    acc.at[pl.ds(s * RPS, RPS)])
    pltpu.sync_copy(ones_hbm, ones_v)
    plsc.subcore_barrier()

    def body(i, carry):
      base = wid * per_tile + i * SLAB
      pltpu.sync_copy(ei_hbm.at[pl.ds(base, SLAB)], idx_v)
      for j in range(SLAB):
        pltpu.sync_copy(ones_v, acc.at[idx_v.at[j, 1]], add=True)
      return carry

    lax.fori_loop(0, outer, body, 0)
    plsc.subcore_barrier()
    pltpu.sync_copy(acc.at[pl.ds(s * RPS, RPS)],
                    out_hbm.at[c, pl.ds(s * RPS, RPS)])

  return deg_kernel(ei_chunks, ones_rows, zeros_acc)


def _sc_edge_agg(ei_chunks, g, zeros_acc):
  nchunks = ei_chunks.shape[0]
  per_tile = nchunks // (NC * NS)
  outer = per_tile // SLAB

  @functools.partial(
      pl.kernel,
      mesh=_sc_mesh(),
      out_type=jax.ShapeDtypeStruct((NC, NP, CP), jnp.float32),
      compiler_params=pltpu.CompilerParams(use_tc_tiling_on_sc=False),
      scratch_types=[
          pltpu.VMEM((SLAB, 2, CHUNK), jnp.int32),
          pltpu.VMEM((SLAB, CHUNK, CP), jnp.float32),
          pltpu.VMEM_SHARED((NP, CP), jnp.float32),
          pltpu.SemaphoreType.DMA,
      ],
  )
  def agg_kernel(ei_hbm, g_hbm, zeros_hbm, out_hbm, idx_v, rows_v, acc, sem):
    c = lax.axis_index("c")
    s = lax.axis_index("s")
    wid = c * NS + s
    pltpu.sync_copy(zeros_hbm.at[pl.ds(s * RPS, RPS)],
                    acc.at[pl.ds(s * RPS, RPS)])
    plsc.subcore_barrier()

    def body(i, carry):
      base = wid * per_tile + i * SLAB
      pltpu.sync_copy(ei_hbm.at[pl.ds(base, SLAB)], idx_v)
      copies = [
          pltpu.async_copy(g_hbm.at[idx_v.at[j, 0]], rows_v.at[j], sem)
          for j in range(SLAB)
      ]
      for j in range(SLAB):
        copies[j].wait()
      for j in range(SLAB):
        pltpu.sync_copy(rows_v.at[j], acc.at[idx_v.at[j, 1]], add=True)
      return carry

    lax.fori_loop(0, outer, body, 0)
    plsc.subcore_barrier()
    pltpu.sync_copy(acc.at[pl.ds(s * RPS, RPS)],
                    out_hbm.at[c, pl.ds(s * RPS, RPS)])

  return agg_kernel(ei_chunks, g, zeros_acc)


def _fs(shape):
  return pl.BlockSpec(shape, lambda: (0,) * len(shape))


def _bn(h, g, b, m, v):
  return (h - m) * lax.rsqrt(v + EPS) * g + b


def _tc_input(x, Win, b_in, cnt, W0):
  def body(x_ref, win_ref, bin_ref, cnt_ref, w0_ref, h_ref, g_ref, dinv_ref):
    h = jnp.dot(x_ref[...], win_ref[...],
                preferred_element_type=jnp.float32) + bin_ref[...]
    deg = 1.0 + cnt_ref[0, :N, :] + cnt_ref[1, :N, :]
    dinv = lax.rsqrt(deg)
    dinv_ref[...] = dinv
    hW = jnp.dot(h, w0_ref[...], preferred_element_type=jnp.float32)
    gg = hW * dinv[:, :CH]
    g_ref[...] = jnp.concatenate(
        [gg, jnp.zeros((N, CP - CH), jnp.float32)], axis=1)
    h_ref[...] = h

  return pl.pallas_call(
      body,
      out_shape=[
          jax.ShapeDtypeStruct((N, CH), jnp.float32),
          jax.ShapeDtypeStruct((N, CP), jnp.float32),
          jax.ShapeDtypeStruct((N, CP), jnp.float32),
      ],
      in_specs=[_fs(x.shape), _fs(Win.shape), _fs(b_in.shape),
                _fs(cnt.shape), _fs(W0.shape)],
      out_specs=[_fs((N, CH)), _fs((N, CP)), _fs((N, CP))],
  )(x, Win, b_in, cnt, W0)


def _tc_attention(h, p):
  nblk = N // BQ

  def body(hq_ref, h_ref, wq, bq, wk, bk, wv, bv, wo, bo,
           g2, b2, m2, v2, out_ref, k_s, v_s):
    i = pl.program_id(0)

    @pl.when(i == 0)
    def _():
      hf = h_ref[...]
      k_s[...] = jnp.dot(hf, wk[...],
                         preferred_element_type=jnp.float32) + bk[...]
      v_s[...] = jnp.dot(hf, wv[...],
                         preferred_element_type=jnp.float32) + bv[...]

    hq = hq_ref[...]
    q = jnp.dot(hq, wq[...], preferred_element_type=jnp.float32) + bq[...]
    s = lax.dot_general(q, k_s[...], (((1,), (1,)), ((), ())),
                        preferred_element_type=jnp.float32)
    s = s * (1.0 / jnp.sqrt(float(CH)))
    mx = jnp.max(s, axis=1, keepdims=True)
    ex = jnp.exp(s - mx)
    den = jnp.sum(ex, axis=1, keepdims=True)
    o = jnp.dot(ex, v_s[...], preferred_element_type=jnp.float32) / den
    hg = jnp.dot(o, wo[...], preferred_element_type=jnp.float32) \
        + bo[...] + hq
    out_ref[...] = _bn(hg, g2[...], b2[...], m2[...], v2[...])

  params = [p['Wq'], p['bq'], p['Wk'], p['bk'], p['Wv'], p['bv'],
            p['Wo'], p['bo'], p['bn2_g'], p['bn2_b'], p['bn2_m'], p['bn2_v']]

  def cspec(a):
    sh = a.shape
    return pl.BlockSpec(sh, lambda i: (0,) * len(sh))

  return pl.pallas_call(
      body,
      grid=(nblk,),
      out_shape=jax.ShapeDtypeStruct((N, CH), jnp.float32),
      in_specs=[pl.BlockSpec((BQ, CH), lambda i: (i, 0)),
                pl.BlockSpec((N, CH), lambda i: (0, 0))] +
               [cspec(a) for a in params],
      out_specs=pl.BlockSpec((BQ, CH), lambda i: (i, 0)),
      scratch_shapes=[pltpu.VMEM((N, CH), jnp.float32),
                      pltpu.VMEM((N, CH), jnp.float32)],
  )(h, h, *params)


def _tc_combine(h, part, hg, dinv, p, lng, lnb, W_next, b_next, last):
  def body(h_ref, part_ref, hg_ref, dinv_ref, gw, gb,
           g1, b1_, m1, v1, w1, bb1, w2, bb2,
           g3, b3_, m3, v3, lng_ref, lnb_ref, wn, bn_, *outs):
    h0 = h_ref[...]
    dinv = dinv_ref[...][:, :CH]
    hW = jnp.dot(h0, gw[...], preferred_element_type=jnp.float32)
    e = part_ref[0, :N, :CH] + part_ref[1, :N, :CH]
    hl = e * dinv + hW * (dinv * dinv) + gb[...] + h0
    hl = _bn(hl, g1[...], b1_[...], m1[...], v1[...])
    out = hl + hg_ref[...]
    mlp = jnp.maximum(
        jnp.dot(out, w1[...], preferred_element_type=jnp.float32) + bb1[...],
        0.0)
    mlp = jnp.dot(mlp, w2[...], preferred_element_type=jnp.float32) + bb2[...]
    out = out + mlp
    out = _bn(out, g3[...], b3_[...], m3[...], v3[...])
    mu = jnp.sum(out, axis=1, keepdims=True) * (1.0 / CH)
    d = out - mu
    var = jnp.sum(d * d, axis=1, keepdims=True) * (1.0 / CH)
    hn = jnp.maximum(d * lax.rsqrt(var + EPS) * lng_ref[...] + lnb_ref[...],
                     0.0)
    if last:
      logits = jnp.dot(hn, wn[...], preferred_element_type=jnp.float32) \
          + bn_[...]
      mx = jnp.max(logits, axis=1, keepdims=True)
      z = logits - mx
      lse = jnp.log(jnp.sum(jnp.exp(z), axis=1, keepdims=True))
      outs[0][...] = z - lse
    else:
      outs[0][...] = hn
      gg = jnp.dot(hn, wn[...], preferred_element_type=jnp.float32) * dinv
      outs[1][...] = jnp.concatenate(
          [gg, jnp.zeros((N, CP - CH), jnp.float32)], axis=1)

  ins = [h, part, hg, dinv, p['gcn_W'], p['gcn_b'],
         p['bn1_g'], p['bn1_b'], p['bn1_m'], p['bn1_v'],
         p['W1'], p['b1'], p['W2'], p['b2'],
         p['bn3_g'], p['bn3_b'], p['bn3_m'], p['bn3_v'],
         lng, lnb, W_next, b_next]
  if last:
    out_shape = [jax.ShapeDtypeStruct((N, W_next.shape[1]), jnp.float32)]
  else:
    out_shape = [jax.ShapeDtypeStruct((N, CH), jnp.float32),
                 jax.ShapeDtypeStruct((N, CP), jnp.float32)]
  return pl.pallas_call(
      body,
      out_shape=out_shape,
      in_specs=[_fs(a.shape) for a in ins],
      out_specs=[_fs(o.shape) for o in out_shape],
  )(*ins)


def kernel(x, edge_index, Win, b_in, params, Wout, b_out, ln_g, ln_b):
  ei = edge_index.astype(jnp.int32)
  E = ei.shape[1]
  group = NC * NS * CHUNK * SLAB
  EP = ((E + group - 1) // group) * group
  pad = EP - E
  src = jnp.concatenate([ei[0], jnp.zeros((pad,), jnp.int32)])
  dst = jnp.concatenate([ei[1], jnp.full((pad,), N, jnp.int32)])
  ei_chunks = jnp.stack(
      [src.reshape(-1, CHUNK), dst.reshape(-1, CHUNK)], axis=1)

  ones_rows = jnp.ones((CHUNK, CP), jnp.float32)
  zeros_acc = jnp.zeros((NP, CP), jnp.float32)

  def r2(v):
    return v.reshape(1, -1)

  cnt = _sc_degree(ei_chunks, ones_rows, zeros_acc)
  h0, g0, dinv = _tc_input(x, Win, r2(b_in), cnt, params['gcn_W'][0])

  h = h0
  g = g0
  for l in range(2):
    p = dict((k, v[l]) for k, v in params.items())
    for k in list(p):
      if p[k].ndim == 1:
        p[k] = r2(p[k])
    part = _sc_edge_agg(ei_chunks, g, zeros_acc)
    hg = _tc_attention(h, p)
    last = (l == 1)
    if last:
      res = _tc_combine(h, part, hg, dinv, p, r2(ln_g[l]), r2(ln_b[l]),
                        Wout, r2(b_out), True)
      return res[0]
    nxt = _tc_combine(h, part, hg, dinv, p, r2(ln_g[l]), r2(ln_b[l]),
                      params['gcn_W'][l + 1], r2(params['gcn_b'][l + 1]),
                      False)
    h, g = nxt[0], nxt[1]


# speedup vs baseline: 16.1560x; 1.2321x over previous
import functools
import jax
import jax.numpy as jnp
from jax import lax
from jax.experimental import pallas as pl
from jax.experimental.pallas import tpu as pltpu
from jax.experimental.pallas import tpu_sc as plsc

N = 10000
CH = 12
CP = 16
NP = N + 112
RPS = NP // 16
EPS = 1e-5
NC, NS = 2, 16
CHUNK = 128
SLAB = 8
BQ = 400

_sc_mesh = functools.partial(
    plsc.VectorSubcoreMesh, core_axis_name="c", subcore_axis_name="s")


def _sc_degree(ei_chunks, ones_rows, zeros_acc):
  nchunks = ei_chunks.shape[0]
  per_tile = nchunks // (NC * NS)
  outer = per_tile // SLAB

  @functools.partial(
      pl.kernel,
      mesh=_sc_mesh(),
      out_type=jax.ShapeDtypeStruct((NC, NP, CP), jnp.float32),
      compiler_params=pltpu.CompilerParams(use_tc_tiling_on_sc=False),
      scratch_types=[
          pltpu.VMEM((SLAB, 2, CHUNK), jnp.int32),
          pltpu.VMEM((CHUNK, CP), jnp.float32),
          pltpu.VMEM_SHARED((NP, CP), jnp.float32),
      ],
  )
  def deg_kernel(ei_hbm, ones_hbm, zeros_hbm, out_hbm, idx_v, ones_v, acc):
    c = lax.axis_index("c")
    s = lax.axis_index("s")
    wid = c * NS + s
    pltpu.sync_copy(zeros_hbm.at[pl.ds(s * RPS, RPS)],
                    acc.at[pl.ds(s * RPS, RPS)])
    pltpu.sync_copy(ones_hbm, ones_v)
    plsc.subcore_barrier()

    def body(i, carry):
      base = wid * per_tile + i * SLAB
      pltpu.sync_copy(ei_hbm.at[pl.ds(base, SLAB)], idx_v)
      for j in range(SLAB):
        pltpu.sync_copy(ones_v, acc.at[idx_v.at[j, 1]], add=True)
      return carry

    lax.fori_loop(0, outer, body, 0)
    plsc.subcore_barrier()
    pltpu.sync_copy(acc.at[pl.ds(s * RPS, RPS)],
                    out_hbm.at[c, pl.ds(s * RPS, RPS)])

  return deg_kernel(ei_chunks, ones_rows, zeros_acc)


def _sc_edge_agg(ei_chunks, g, zeros_acc):
  nchunks = ei_chunks.shape[0]
  per_tile = nchunks // (NC * NS)
  outer = per_tile // SLAB

  @functools.partial(
      pl.kernel,
      mesh=_sc_mesh(),
      out_type=jax.ShapeDtypeStruct((NC, NP, CP), jnp.float32),
      compiler_params=pltpu.CompilerParams(use_tc_tiling_on_sc=False),
      scratch_types=[
          pltpu.VMEM((SLAB, 2, CHUNK), jnp.int32),
          pltpu.VMEM((SLAB, CHUNK, CP), jnp.float32),
          pltpu.VMEM_SHARED((NP, CP), jnp.float32),
          pltpu.SemaphoreType.DMA,
      ],
  )
  def agg_kernel(ei_hbm, g_hbm, zeros_hbm, out_hbm, idx_v, rows_v, acc, sem):
    c = lax.axis_index("c")
    s = lax.axis_index("s")
    wid = c * NS + s
    pltpu.sync_copy(zeros_hbm.at[pl.ds(s * RPS, RPS)],
                    acc.at[pl.ds(s * RPS, RPS)])
    plsc.subcore_barrier()

    def body(i, carry):
      base = wid * per_tile + i * SLAB
      pltpu.sync_copy(ei_hbm.at[pl.ds(base, SLAB)], idx_v)
      copies = [
          pltpu.async_copy(g_hbm.at[idx_v.at[j, 0]], rows_v.at[j], sem)
          for j in range(SLAB)
      ]
      for j in range(SLAB):
        copies[j].wait()
      for j in range(SLAB):
        pltpu.sync_copy(rows_v.at[j], acc.at[idx_v.at[j, 1]], add=True)
      return carry

    lax.fori_loop(0, outer, body, 0)
    plsc.subcore_barrier()
    pltpu.sync_copy(acc.at[pl.ds(s * RPS, RPS)],
                    out_hbm.at[c, pl.ds(s * RPS, RPS)])

  return agg_kernel(ei_chunks, g, zeros_acc)


def _fs(shape):
  return pl.BlockSpec(shape, lambda: (0,) * len(shape))


def _bn(h, g, b, m, v):
  return (h - m) * lax.rsqrt(v + EPS) * g + b


def _tc_input(x, Win, b_in, cnt, W0):
  def body(x_ref, win_ref, bin_ref, cnt_ref, w0_ref, h_ref, g_ref, dinv_ref):
    h = jnp.dot(x_ref[...], win_ref[...],
                preferred_element_type=jnp.float32) + bin_ref[...]
    deg = 1.0 + cnt_ref[0, :N, :] + cnt_ref[1, :N, :]
    dinv = lax.rsqrt(deg)
    dinv_ref[...] = dinv
    hW = jnp.dot(h, w0_ref[...], preferred_element_type=jnp.float32)
    gg = hW * dinv[:, :CH]
    g_ref[...] = jnp.concatenate(
        [gg, jnp.zeros((N, CP - CH), jnp.float32)], axis=1)
    h_ref[...] = h

  return pl.pallas_call(
      body,
      out_shape=[
          jax.ShapeDtypeStruct((N, CH), jnp.float32),
          jax.ShapeDtypeStruct((N, CP), jnp.float32),
          jax.ShapeDtypeStruct((N, CP), jnp.float32),
      ],
      in_specs=[_fs(x.shape), _fs(Win.shape), _fs(b_in.shape),
                _fs(cnt.shape), _fs(W0.shape)],
      out_specs=[_fs((N, CH)), _fs((N, CP)), _fs((N, CP))],
  )(x, Win, b_in, cnt, W0)


def _tc_attention(h, p):
  nblk = N // BQ

  def body(hq_ref, h_ref, wq, bq, wk, bk, wv, bv, wo, bo,
           g2, b2, m2, v2, out_ref, k_s, v_s):
    i = pl.program_id(0)

    @pl.when(i == 0)
    def _():
      hf = h_ref[...]
      k_s[...] = jnp.dot(hf, wk[...],
                         preferred_element_type=jnp.float32) + bk[...]
      v_s[...] = jnp.concatenate(
          [jnp.dot(hf, wv[...], preferred_element_type=jnp.float32)
           + bv[...], jnp.ones((N, 1), jnp.float32)], axis=1)

    hq = hq_ref[...]
    q = (jnp.dot(hq, wq[...], preferred_element_type=jnp.float32)
         + bq[...]) * (1.0 / jnp.sqrt(float(CH)))
    s = lax.dot_general(q, k_s[...], (((1,), (1,)), ((), ())),
                        preferred_element_type=jnp.float32)
    mx = jnp.max(s, axis=1, keepdims=True)
    ex = jnp.exp(s - mx)
    ov = jnp.dot(ex, v_s[...], preferred_element_type=jnp.float32)
    o = ov[:, :CH] / ov[:, CH:CH + 1]
    hg = jnp.dot(o, wo[...], preferred_element_type=jnp.float32) \
        + bo[...] + hq
    out_ref[...] = _bn(hg, g2[...], b2[...], m2[...], v2[...])

  params = [p['Wq'], p['bq'], p['Wk'], p['bk'], p['Wv'], p['bv'],
            p['Wo'], p['bo'], p['bn2_g'], p['bn2_b'], p['bn2_m'], p['bn2_v']]

  def cspec(a):
    sh = a.shape
    return pl.BlockSpec(sh, lambda i: (0,) * len(sh))

  return pl.pallas_call(
      body,
      grid=(nblk,),
      out_shape=jax.ShapeDtypeStruct((N, CH), jnp.float32),
      in_specs=[pl.BlockSpec((BQ, CH), lambda i: (i, 0)),
                pl.BlockSpec((N, CH), lambda i: (0, 0))] +
               [cspec(a) for a in params],
      out_specs=pl.BlockSpec((BQ, CH), lambda i: (i, 0)),
      scratch_shapes=[pltpu.VMEM((N, CH), jnp.float32),
                      pltpu.VMEM((N, CH + 1), jnp.float32)],
  )(h, h, *params)


def _tc_combine(h, part, hg, dinv, p, lng, lnb, W_next, b_next, last):
  def body(h_ref, part_ref, hg_ref, dinv_ref, gw, gb,
           g1, b1_, m1, v1, w1, bb1, w2, bb2,
           g3, b3_, m3, v3, lng_ref, lnb_ref, wn, bn_, *outs):
    h0 = h_ref[...]
    dinv = dinv_ref[...][:, :CH]
    hW = jnp.dot(h0, gw[...], preferred_element_type=jnp.float32)
    e = part_ref[0, :N, :CH] + part_ref[1, :N, :CH]
    hl = e * dinv + hW * (dinv * dinv) + gb[...] + h0
    hl = _bn(hl, g1[...], b1_[...], m1[...], v1[...])
    out = hl + hg_ref[...]
    mlp = jnp.maximum(
        jnp.dot(out, w1[...], preferred_element_type=jnp.float32) + bb1[...],
        0.0)
    mlp = jnp.dot(mlp, w2[...], preferred_element_type=jnp.float32) + bb2[...]
    out = out + mlp
    out = _bn(out, g3[...], b3_[...], m3[...], v3[...])
    mu = jnp.sum(out, axis=1, keepdims=True) * (1.0 / CH)
    d = out - mu
    var = jnp.sum(d * d, axis=1, keepdims=True) * (1.0 / CH)
    hn = jnp.maximum(d * lax.rsqrt(var + EPS) * lng_ref[...] + lnb_ref[...],
                     0.0)
    if last:
      logits = jnp.dot(hn, wn[...], preferred_element_type=jnp.float32) \
          + bn_[...]
      mx = jnp.max(logits, axis=1, keepdims=True)
      z = logits - mx
      lse = jnp.log(jnp.sum(jnp.exp(z), axis=1, keepdims=True))
      outs[0][...] = z - lse
    else:
      outs[0][...] = hn
      gg = jnp.dot(hn, wn[...], preferred_element_type=jnp.float32) * dinv
      outs[1][...] = jnp.concatenate(
          [gg, jnp.zeros((N, CP - CH), jnp.float32)], axis=1)

  ins = [h, part, hg, dinv, p['gcn_W'], p['gcn_b'],
         p['bn1_g'], p['bn1_b'], p['bn1_m'], p['bn1_v'],
         p['W1'], p['b1'], p['W2'], p['b2'],
         p['bn3_g'], p['bn3_b'], p['bn3_m'], p['bn3_v'],
         lng, lnb, W_next, b_next]
  if last:
    out_shape = [jax.ShapeDtypeStruct((N, W_next.shape[1]), jnp.float32)]
  else:
    out_shape = [jax.ShapeDtypeStruct((N, CH), jnp.float32),
                 jax.ShapeDtypeStruct((N, CP), jnp.float32)]
  return pl.pallas_call(
      body,
      out_shape=out_shape,
      in_specs=[_fs(a.shape) for a in ins],
      out_specs=[_fs(o.shape) for o in out_shape],
  )(*ins)


def kernel(x, edge_index, Win, b_in, params, Wout, b_out, ln_g, ln_b):
  ei = edge_index.astype(jnp.int32)
  E = ei.shape[1]
  group = NC * NS * CHUNK * SLAB
  EP = ((E + group - 1) // group) * group
  pad = EP - E
  src = jnp.concatenate([ei[0], jnp.zeros((pad,), jnp.int32)])
  dst = jnp.concatenate([ei[1], jnp.full((pad,), N, jnp.int32)])
  ei_chunks = jnp.stack(
      [src.reshape(-1, CHUNK), dst.reshape(-1, CHUNK)], axis=1)

  ones_rows = jnp.ones((CHUNK, CP), jnp.float32)
  zeros_acc = jnp.zeros((NP, CP), jnp.float32)

  def r2(v):
    return v.reshape(1, -1)

  cnt = _sc_degree(ei_chunks, ones_rows, zeros_acc)
  h0, g0, dinv = _tc_input(x, Win, r2(b_in), cnt, params['gcn_W'][0])

  h = h0
  g = g0
  for l in range(2):
    p = dict((k, v[l]) for k, v in params.items())
    for k in list(p):
      if p[k].ndim == 1:
        p[k] = r2(p[k])
    part = _sc_edge_agg(ei_chunks, g, zeros_acc)
    hg = _tc_attention(h, p)
    last = (l == 1)
    if last:
      res = _tc_combine(h, part, hg, dinv, p, r2(ln_g[l]), r2(ln_b[l]),
                        Wout, r2(b_out), True)
      return res[0]
    nxt = _tc_combine(h, part, hg, dinv, p, r2(ln_g[l]), r2(ln_b[l]),
                      params['gcn_W'][l + 1], r2(params['gcn_b'][l + 1]),
                      False)
    h, g = nxt[0], nxt[1]


# speedup vs baseline: 16.1860x; 1.0019x over previous
import functools
import jax
import jax.numpy as jnp
from jax import lax
from jax.experimental import pallas as pl
from jax.experimental.pallas import tpu as pltpu
from jax.experimental.pallas import tpu_sc as plsc

N = 10000
CH = 12
CP = 16
NP = N + 112
RPS = NP // 16
EPS = 1e-5
NC, NS = 2, 16
CHUNK = 128
SLAB = 8
BQ = 400

_sc_mesh = functools.partial(
    plsc.VectorSubcoreMesh, core_axis_name="c", subcore_axis_name="s")


def _sc_degree(ei_chunks, ones_rows, zeros_acc):
  nchunks = ei_chunks.shape[0]
  per_tile = nchunks // (NC * NS)
  outer = per_tile // SLAB

  @functools.partial(
      pl.kernel,
      mesh=_sc_mesh(),
      out_type=jax.ShapeDtypeStruct((NC, NP, CP), jnp.float32),
      compiler_params=pltpu.CompilerParams(use_tc_tiling_on_sc=False),
      scratch_types=[
          pltpu.VMEM((SLAB, 2, CHUNK), jnp.int32),
          pltpu.VMEM((CHUNK, CP), jnp.float32),
          pltpu.VMEM_SHARED((NP, CP), jnp.float32),
      ],
  )
  def deg_kernel(ei_hbm, ones_hbm, zeros_hbm, out_hbm, idx_v, ones_v, acc):
    c = lax.axis_index("c")
    s = lax.axis_index("s")
    wid = c * NS + s
    pltpu.sync_copy(zeros_hbm.at[pl.ds(s * RPS, RPS)],
                    acc.at[pl.ds(s * RPS, RPS)])
    pltpu.sync_copy(ones_hbm, ones_v)
    plsc.subcore_barrier()

    def body(i, carry):
      base = wid * per_tile + i * SLAB
      pltpu.sync_copy(ei_hbm.at[pl.ds(base, SLAB)], idx_v)
      for j in range(SLAB):
        pltpu.sync_copy(ones_v, acc.at[idx_v.at[j, 1]], add=True)
      return carry

    lax.fori_loop(0, outer, body, 0)
    plsc.subcore_barrier()
    pltpu.sync_copy(acc.at[pl.ds(s * RPS, RPS)],
                    out_hbm.at[c, pl.ds(s * RPS, RPS)])

  return deg_kernel(ei_chunks, ones_rows, zeros_acc)


def _sc_edge_agg(ei_chunks, g, zeros_acc):
  nchunks = ei_chunks.shape[0]
  per_tile = nchunks // (NC * NS)
  outer = per_tile // SLAB

  @functools.partial(
      pl.kernel,
      mesh=_sc_mesh(),
      out_type=jax.ShapeDtypeStruct((NC, NP, CP), jnp.float32),
      compiler_params=pltpu.CompilerParams(use_tc_tiling_on_sc=False),
      scratch_types=[
          pltpu.VMEM((SLAB, 2, CHUNK), jnp.int32),
          pltpu.VMEM((SLAB, CHUNK, CP), jnp.float32),
          pltpu.VMEM_SHARED((NP, CP), jnp.float32),
          pltpu.VMEM_SHARED((NP, CP), jnp.float32),
          pltpu.SemaphoreType.DMA,
      ],
  )
  def agg_kernel(ei_hbm, g_hbm, zeros_hbm, out_hbm, idx_v, rows_v, acc,
                 g_sh, sem):
    c = lax.axis_index("c")
    s = lax.axis_index("s")
    wid = c * NS + s
    pltpu.sync_copy(g_hbm.at[pl.ds(s * RPS, RPS)],
                    g_sh.at[pl.ds(s * RPS, RPS)])
    pltpu.sync_copy(zeros_hbm.at[pl.ds(s * RPS, RPS)],
                    acc.at[pl.ds(s * RPS, RPS)])
    plsc.subcore_barrier()

    def body(i, carry):
      base = wid * per_tile + i * SLAB
      pltpu.sync_copy(ei_hbm.at[pl.ds(base, SLAB)], idx_v)
      copies = [
          pltpu.async_copy(g_sh.at[idx_v.at[j, 0]], rows_v.at[j], sem)
          for j in range(SLAB)
      ]
      for j in range(SLAB):
        copies[j].wait()
      for j in range(SLAB):
        pltpu.sync_copy(rows_v.at[j], acc.at[idx_v.at[j, 1]], add=True)
      return carry

    lax.fori_loop(0, outer, body, 0)
    plsc.subcore_barrier()
    pltpu.sync_copy(acc.at[pl.ds(s * RPS, RPS)],
                    out_hbm.at[c, pl.ds(s * RPS, RPS)])

  return agg_kernel(ei_chunks, g, zeros_acc)


def _fs(shape):
  return pl.BlockSpec(shape, lambda: (0,) * len(shape))


def _bn(h, g, b, m, v):
  return (h - m) * lax.rsqrt(v + EPS) * g + b


def _pad_g(gg):
  gg = jnp.concatenate([gg, jnp.zeros((N, CP - CH), jnp.float32)], axis=1)
  return jnp.concatenate([gg, jnp.zeros((NP - N, CP), jnp.float32)], axis=0)


def _tc_input(x, Win, b_in, cnt, W0):
  def body(x_ref, win_ref, bin_ref, cnt_ref, w0_ref, h_ref, g_ref, dinv_ref):
    h = jnp.dot(x_ref[...], win_ref[...],
                preferred_element_type=jnp.float32) + bin_ref[...]
    deg = 1.0 + cnt_ref[0, :N, :] + cnt_ref[1, :N, :]
    dinv = lax.rsqrt(deg)
    dinv_ref[...] = dinv
    hW = jnp.dot(h, w0_ref[...], preferred_element_type=jnp.float32)
    g_ref[...] = _pad_g(hW * dinv[:, :CH])
    h_ref[...] = h

  return pl.pallas_call(
      body,
      out_shape=[
          jax.ShapeDtypeStruct((N, CH), jnp.float32),
          jax.ShapeDtypeStruct((NP, CP), jnp.float32),
          jax.ShapeDtypeStruct((N, CP), jnp.float32),
      ],
      in_specs=[_fs(x.shape), _fs(Win.shape), _fs(b_in.shape),
                _fs(cnt.shape), _fs(W0.shape)],
      out_specs=[_fs((N, CH)), _fs((NP, CP)), _fs((N, CP))],
  )(x, Win, b_in, cnt, W0)


def _tc_attention(h, p):
  nblk = N // BQ

  def body(hq_ref, h_ref, wq, bq, wk, bk, wv, bv, wo, bo,
           g2, b2, m2, v2, out_ref, k_s, v_s):
    i = pl.program_id(0)

    @pl.when(i == 0)
    def _():
      hf = h_ref[...]
      k_s[...] = jnp.dot(hf, wk[...],
                         preferred_element_type=jnp.float32) + bk[...]
      v_s[...] = jnp.concatenate(
          [jnp.dot(hf, wv[...], preferred_element_type=jnp.float32)
           + bv[...], jnp.ones((N, 1), jnp.float32)], axis=1)

    hq = hq_ref[...]
    q = (jnp.dot(hq, wq[...], preferred_element_type=jnp.float32)
         + bq[...]) * (1.0 / jnp.sqrt(float(CH)))
    s = lax.dot_general(q, k_s[...], (((1,), (1,)), ((), ())),
                        preferred_element_type=jnp.float32)
    mx = jnp.max(s, axis=1, keepdims=True)
    ex = jnp.exp(s - mx)
    ov = jnp.dot(ex, v_s[...], preferred_element_type=jnp.float32)
    o = ov[:, :CH] / ov[:, CH:CH + 1]
    hg = jnp.dot(o, wo[...], preferred_element_type=jnp.float32) \
        + bo[...] + hq
    out_ref[...] = _bn(hg, g2[...], b2[...], m2[...], v2[...])

  params = [p['Wq'], p['bq'], p['Wk'], p['bk'], p['Wv'], p['bv'],
            p['Wo'], p['bo'], p['bn2_g'], p['bn2_b'], p['bn2_m'], p['bn2_v']]

  def cspec(a):
    sh = a.shape
    return pl.BlockSpec(sh, lambda i: (0,) * len(sh))

  return pl.pallas_call(
      body,
      grid=(nblk,),
      out_shape=jax.ShapeDtypeStruct((N, CH), jnp.float32),
      in_specs=[pl.BlockSpec((BQ, CH), lambda i: (i, 0)),
                pl.BlockSpec((N, CH), lambda i: (0, 0))] +
               [cspec(a) for a in params],
      out_specs=pl.BlockSpec((BQ, CH), lambda i: (i, 0)),
      scratch_shapes=[pltpu.VMEM((N, CH), jnp.float32),
                      pltpu.VMEM((N, CH + 1), jnp.float32)],
  )(h, h, *params)


def _tc_combine(h, part, hg, dinv, p, lng, lnb, W_next, b_next, last):
  def body(h_ref, part_ref, hg_ref, dinv_ref, gw, gb,
           g1, b1_, m1, v1, w1, bb1, w2, bb2,
           g3, b3_, m3, v3, lng_ref, lnb_ref, wn, bn_, *outs):
    h0 = h_ref[...]
    dinv = dinv_ref[...][:, :CH]
    hW = jnp.dot(h0, gw[...], preferred_element_type=jnp.float32)
    e = part_ref[0, :N, :CH] + part_ref[1, :N, :CH]
    hl = e * dinv + hW * (dinv * dinv) + gb[...] + h0
    hl = _bn(hl, g1[...], b1_[...], m1[...], v1[...])
    out = hl + hg_ref[...]
    mlp = jnp.maximum(
        jnp.dot(out, w1[...], preferred_element_type=jnp.float32) + bb1[...],
        0.0)
    mlp = jnp.dot(mlp, w2[...], preferred_element_type=jnp.float32) + bb2[...]
    out = out + mlp
    out = _bn(out, g3[...], b3_[...], m3[...], v3[...])
    mu = jnp.sum(out, axis=1, keepdims=True) * (1.0 / CH)
    d = out - mu
    var = jnp.sum(d * d, axis=1, keepdims=True) * (1.0 / CH)
    hn = jnp.maximum(d * lax.rsqrt(var + EPS) * lng_ref[...] + lnb_ref[...],
                     0.0)
    if last:
      logits = jnp.dot(hn, wn[...], preferred_element_type=jnp.float32) \
          + bn_[...]
      mx = jnp.max(logits, axis=1, keepdims=True)
      z = logits - mx
      lse = jnp.log(jnp.sum(jnp.exp(z), axis=1, keepdims=True))
      outs[0][...] = z - lse
    else:
      outs[0][...] = hn
      gg = jnp.dot(hn, wn[...], preferred_element_type=jnp.float32) * dinv
      outs[1][...] = _pad_g(gg)

  ins = [h, part, hg, dinv, p['gcn_W'], p['gcn_b'],
         p['bn1_g'], p['bn1_b'], p['bn1_m'], p['bn1_v'],
         p['W1'], p['b1'], p['W2'], p['b2'],
         p['bn3_g'], p['bn3_b'], p['bn3_m'], p['bn3_v'],
         lng, lnb, W_next, b_next]
  if last:
    out_shape = [jax.ShapeDtypeStruct((N, W_next.shape[1]), jnp.float32)]
  else:
    out_shape = [jax.ShapeDtypeStruct((N, CH), jnp.float32),
                 jax.ShapeDtypeStruct((NP, CP), jnp.float32)]
  return pl.pallas_call(
      body,
      out_shape=out_shape,
      in_specs=[_fs(a.shape) for a in ins],
      out_specs=[_fs(o.shape) for o in out_shape],
  )(*ins)


def kernel(x, edge_index, Win, b_in, params, Wout, b_out, ln_g, ln_b):
  ei = edge_index.astype(jnp.int32)
  E = ei.shape[1]
  group = NC * NS * CHUNK * SLAB
  EP = ((E + group - 1) // group) * group
  pad = EP - E
  src = jnp.concatenate([ei[0], jnp.zeros((pad,), jnp.int32)])
  dst = jnp.concatenate([ei[1], jnp.full((pad,), N, jnp.int32)])
  ei_chunks = jnp.stack(
      [src.reshape(-1, CHUNK), dst.reshape(-1, CHUNK)], axis=1)

  ones_rows = jnp.ones((CHUNK, CP), jnp.float32)
  zeros_acc = jnp.zeros((NP, CP), jnp.float32)

  def r2(v):
    return v.reshape(1, -1)

  cnt = _sc_degree(ei_chunks, ones_rows, zeros_acc)
  h0, g0, dinv = _tc_input(x, Win, r2(b_in), cnt, params['gcn_W'][0])

  h = h0
  g = g0
  for l in range(2):
    p = dict((k, v[l]) for k, v in params.items())
    for k in list(p):
      if p[k].ndim == 1:
        p[k] = r2(p[k])
    part = _sc_edge_agg(ei_chunks, g, zeros_acc)
    hg = _tc_attention(h, p)
    last = (l == 1)
    if last:
      res = _tc_combine(h, part, hg, dinv, p, r2(ln_g[l]), r2(ln_b[l]),
                        Wout, r2(b_out), True)
      return res[0]
    nxt = _tc_combine(h, part, hg, dinv, p, r2(ln_g[l]), r2(ln_b[l]),
                      params['gcn_W'][l + 1], r2(params['gcn_b'][l + 1]),
                      False)
    h, g = nxt[0], nxt[1]
